# Initial kernel scaffold; baseline (speedup 1.0000x reference)
import functools

import jax
import jax.numpy as jnp
from jax import lax
from jax.experimental import pallas as pl
from jax.experimental.pallas import tpu as pltpu
from jax.experimental.pallas import tpu_sc as plsc

N_NODES = 10000
D_FEAT = 128
N_EDGES = 320000
DEPTH = 3
BASEALPHA = 1.0
A_C = 1.0
B_C = 1.0
L_LO = -1.0
R_HI = 1.0

NC = 2
NS = 16
NW = NC * NS
EPW = N_EDGES // NW
CH = 80
NCHUNK = EPW // CH
RPT = N_NODES // NS
ZR = 125
DEGW = 16

_mesh = plsc.VectorSubcoreMesh(core_axis_name="c", subcore_axis_name="s")


@functools.partial(
    pl.kernel,
    mesh=_mesh,
    out_type=jax.ShapeDtypeStruct((NC, N_NODES, DEGW), jnp.float32),
    scratch_types=[
        pltpu.VMEM((CH,), jnp.int32),
        pltpu.VMEM((CH, DEGW), jnp.float32),
        pltpu.VMEM((RPT, DEGW), jnp.float32),
        pltpu.VMEM_SHARED((N_NODES, DEGW), jnp.float32),
    ],
)
def _deg_kernel(ei_hbm, out_hbm, rowv, onesb, zbuf, acc):
    cid = lax.axis_index("c")
    sid = lax.axis_index("s")
    wid = sid * NC + cid

    def fill(i, carry):
        onesb[i, :] = jnp.full((DEGW,), 1.0, jnp.float32)
        return carry

    lax.fori_loop(0, CH, fill, 0)

    def zfill(i, carry):
        zbuf[i, :] = jnp.zeros((DEGW,), jnp.float32)
        return carry

    lax.fori_loop(0, RPT, zfill, 0)
    pltpu.sync_copy(zbuf, acc.at[pl.ds(sid * RPT, RPT)])
    plsc.subcore_barrier()

    ebase = wid * EPW

    def chunk(j, carry):
        pltpu.sync_copy(ei_hbm.at[0, pl.ds(ebase + j * CH, CH)], rowv)
        pltpu.sync_copy(onesb, acc.at[rowv], add=True)
        return carry

    lax.fori_loop(0, NCHUNK, chunk, 0)
    plsc.subcore_barrier()
    pltpu.sync_copy(
        acc.at[pl.ds(sid * RPT, RPT)], out_hbm.at[cid, pl.ds(sid * RPT, RPT)]
    )


@functools.partial(
    pl.kernel,
    mesh=_mesh,
    out_type=jax.ShapeDtypeStruct((NC, N_NODES, D_FEAT), jnp.float32),
    scratch_types=[
        pltpu.VMEM((CH,), jnp.int32),
        pltpu.VMEM((CH,), jnp.int32),
        pltpu.VMEM((CH,), jnp.float32),
        pltpu.VMEM((CH, D_FEAT), jnp.float32),
        pltpu.VMEM((ZR, D_FEAT), jnp.float32),
        pltpu.VMEM_SHARED((N_NODES, D_FEAT), jnp.float32),
        pltpu.SemaphoreType.DMA,
    ],
)
def _spmm_kernel(u_hbm, ei_hbm, w_hbm, out_hbm, colv, rowv, wv, rows, zbuf, acc, sem):
    cid = lax.axis_index("c")
    sid = lax.axis_index("s")
    wid = sid * NC + cid

    def zfill(i, carry):
        for k in range(D_FEAT // 16):
            zbuf[i, pl.ds(k * 16, 16)] = jnp.zeros((16,), jnp.float32)
        return carry

    lax.fori_loop(0, ZR, zfill, 0)
    for q in range(RPT // ZR):
        pltpu.sync_copy(zbuf, acc.at[pl.ds(sid * RPT + q * ZR, ZR)])
    plsc.subcore_barrier()

    ebase = wid * EPW

    def chunk(j, carry):
        base = ebase + j * CH
        pltpu.sync_copy(ei_hbm.at[1, pl.ds(base, CH)], colv)
        pltpu.sync_copy(ei_hbm.at[0, pl.ds(base, CH)], rowv)
        pltpu.sync_copy(w_hbm.at[pl.ds(base, CH)], wv)
        pltpu.async_copy(u_hbm.at[colv], rows, sem).wait()

        def scale(e, c2):
            bc = lax.broadcast(wv[e], (16,))
            for k in range(D_FEAT // 16):
                rows[e, pl.ds(k * 16, 16)] = rows[e, pl.ds(k * 16, 16)] * bc
            return c2

        lax.fori_loop(0, CH, scale, 0)
        pltpu.sync_copy(rows, acc.at[rowv], add=True)
        return carry

    lax.fori_loop(0, NCHUNK, chunk, 0)
    plsc.subcore_barrier()
    for q in range(RPT // ZR):
        sl = pl.ds(sid * RPT + q * ZR, ZR)
        pltpu.sync_copy(acc.at[sl], out_hbm.at[cid, sl])


_RB = 400
_GRID = N_NODES // _RB


def _prep_body(degp_ref, x_ref, dinv_ref, u0_ref):
    deg = degp_ref[0] + degp_ref[1]
    deg = jnp.where(deg < 0.5, deg + 1.0, deg)
    dinv = lax.rsqrt(deg)
    dinv_ref[...] = dinv
    u0_ref[...] = x_ref[...] * dinv[:, 0:1]


_prep = pl.pallas_call(
    _prep_body,
    grid=(_GRID,),
    in_specs=[
        pl.BlockSpec((NC, _RB, DEGW), lambda i: (0, i, 0)),
        pl.BlockSpec((_RB, D_FEAT), lambda i: (i, 0)),
    ],
    out_specs=[
        pl.BlockSpec((_RB, DEGW), lambda i: (i, 0)),
        pl.BlockSpec((_RB, D_FEAT), lambda i: (i, 0)),
    ],
    out_shape=[
        jax.ShapeDtypeStruct((N_NODES, DEGW), jnp.float32),
        jax.ShapeDtypeStruct((N_NODES, D_FEAT), jnp.float32),
    ],
)


def _combine_body(coef_ref, sp_ref, dinv_ref, yp_ref, yp2_ref, y_ref, u_ref):
    dinv = dinv_ref[...][:, 0:1]
    s = (sp_ref[0] + sp_ref[1]) * dinv
    y = coef_ref[0] * s + coef_ref[1] * yp_ref[...] + coef_ref[2] * yp2_ref[...]
    y_ref[...] = y
    u_ref[...] = y * dinv


_combine = pl.pallas_call(
    _combine_body,
    grid=(_GRID,),
    in_specs=[
        pl.BlockSpec(memory_space=pltpu.SMEM),
        pl.BlockSpec((NC, _RB, D_FEAT), lambda i: (0, i, 0)),
        pl.BlockSpec((_RB, DEGW), lambda i: (i, 0)),
        pl.BlockSpec((_RB, D_FEAT), lambda i: (i, 0)),
        pl.BlockSpec((_RB, D_FEAT), lambda i: (i, 0)),
    ],
    out_specs=[
        pl.BlockSpec((_RB, D_FEAT), lambda i: (i, 0)),
        pl.BlockSpec((_RB, D_FEAT), lambda i: (i, 0)),
    ],
    out_shape=[
        jax.ShapeDtypeStruct((N_NODES, D_FEAT), jnp.float32),
        jax.ShapeDtypeStruct((N_NODES, D_FEAT), jnp.float32),
    ],
)


def _coefficients(alphas):
    al = BASEALPHA * jnp.tanh(alphas)
    rml = R_HI - L_LO
    c1 = ((A_C - B_C) / 2 - (A_C + B_C + 2) / 2 * (L_LO + R_HI) / rml) * al[0]
    c2 = ((A_C + B_C + 2) / rml) * al[0]
    out = [jnp.stack([c2, c1, jnp.zeros_like(c1)])]
    for L in range(2, DEPTH + 1):
        coef_l = 2 * L * (L + A_C + B_C) * (2 * L - 2 + A_C + B_C)
        coef_lm1_1 = (2 * L + A_C + B_C - 1) * (2 * L + A_C + B_C) * (2 * L + A_C + B_C - 2)
        coef_lm1_2 = (2 * L + A_C + B_C - 1) * (A_C**2 - B_C**2)
        coef_lm2 = 2 * (L - 1 + A_C) * (L - 1 + B_C) * (2 * L + A_C + B_C)
        tmp1 = al[L - 1] * (coef_lm1_1 / coef_l)
        tmp2 = al[L - 1] * (coef_lm1_2 / coef_l)
        tmp3 = al[L - 1] * al[L - 2] * (coef_lm2 / coef_l)
        tmp1_2 = tmp1 * (2 / rml)
        tmp2_2 = tmp1 * ((R_HI + L_LO) / rml) + tmp2
        out.append(jnp.stack([tmp1_2, -tmp2_2, -tmp3]))
    return out


def kernel(x, edge_index, edge_attr, alphas):
    degp = _deg_kernel(edge_index)
    dinv16, u = _prep(degp, x)
    coefs = _coefficients(alphas)
    ys = [x]
    yprev2 = x
    for cf in coefs:
        sp = _spmm_kernel(u, edge_index, edge_attr)
        y, u = _combine(cf, sp, dinv16, ys[-1], yprev2)
        yprev2 = ys[-1]
        ys.append(y)
    return jnp.stack(ys, axis=1)


# speedup vs baseline: 5.6354x; 5.6354x over previous
import functools

import jax
import jax.numpy as jnp
from jax import lax
from jax.experimental import pallas as pl
from jax.experimental.pallas import tpu as pltpu
from jax.experimental.pallas import tpu_sc as plsc

N_NODES = 10000
D_FEAT = 128
N_EDGES = 320000
DEPTH = 3
BASEALPHA = 1.0
A_C = 1.0
B_C = 1.0
L_LO = -1.0
R_HI = 1.0

NC = 2
NS = 16
NW = NC * NS
EPW = N_EDGES // NW
CH = 80
NCHUNK = EPW // CH
RPT = N_NODES // NS
ZR = 125
DEGW = 16

def _deg_body(row_hbm, out_hbm, rowv, onesb, zbuf, acc):
    cid = lax.axis_index("c")
    sid = lax.axis_index("s")
    wid = sid * NC + cid

    def fill(i, carry):
        onesb[i, :] = jnp.full((DEGW,), 1.0, jnp.float32)
        return carry

    lax.fori_loop(0, CH, fill, 0)

    def zfill(i, carry):
        zbuf[i, :] = jnp.zeros((DEGW,), jnp.float32)
        return carry

    lax.fori_loop(0, RPT, zfill, 0)
    pltpu.sync_copy(zbuf, acc.at[pl.ds(sid * RPT, RPT)])
    plsc.subcore_barrier()

    ebase = wid * EPW

    def chunk(j, carry):
        pltpu.sync_copy(row_hbm.at[pl.ds(ebase + j * CH, CH)], rowv)
        pltpu.sync_copy(onesb, acc.at[rowv], add=True)
        return carry

    lax.fori_loop(0, NCHUNK, chunk, 0)
    plsc.subcore_barrier()
    pltpu.sync_copy(
        acc.at[pl.ds(sid * RPT, RPT)], out_hbm.at[cid, pl.ds(sid * RPT, RPT)]
    )


def _spmm_body(u_hbm, row_hbm, col_hbm, w_hbm, out_hbm, colv, rowv, wv, rows, zbuf, acc, sem):
    cid = lax.axis_index("c")
    sid = lax.axis_index("s")
    wid = sid * NC + cid

    def zfill(i, carry):
        for k in range(D_FEAT // 16):
            zbuf[i, pl.ds(k * 16, 16)] = jnp.zeros((16,), jnp.float32)
        return carry

    lax.fori_loop(0, ZR, zfill, 0)
    for q in range(RPT // ZR):
        pltpu.sync_copy(zbuf, acc.at[pl.ds(sid * RPT + q * ZR, ZR)])
    plsc.subcore_barrier()

    ebase = wid * EPW

    def chunk(j, carry):
        base = ebase + j * CH
        pltpu.sync_copy(col_hbm.at[pl.ds(base, CH)], colv)
        pltpu.sync_copy(row_hbm.at[pl.ds(base, CH)], rowv)
        pltpu.sync_copy(w_hbm.at[pl.ds(base, CH)], wv)
        pltpu.async_copy(u_hbm.at[colv], rows, sem).wait()

        def scale(e, c2):
            bc = plsc.load_gather(wv, [lax.broadcast(e, (16,))])
            for k in range(D_FEAT // 16):
                rows[e, pl.ds(k * 16, 16)] = rows[e, pl.ds(k * 16, 16)] * bc
            return c2

        lax.fori_loop(0, CH, scale, 0)
        pltpu.sync_copy(rows, acc.at[rowv], add=True)
        return carry

    lax.fori_loop(0, NCHUNK, chunk, 0)
    plsc.subcore_barrier()
    for q in range(RPT // ZR):
        sl = pl.ds(sid * RPT + q * ZR, ZR)
        pltpu.sync_copy(acc.at[sl], out_hbm.at[cid, sl])


@functools.lru_cache(maxsize=None)
def _sc_kernels():
    mesh = plsc.VectorSubcoreMesh(
        core_axis_name="c", subcore_axis_name="s", num_cores=NC, num_subcores=NS
    )
    params = pltpu.CompilerParams(use_tc_tiling_on_sc=False, needs_layout_passes=False)
    deg_kernel = pl.kernel(
        _deg_body,
        mesh=mesh,
        compiler_params=params,
        out_type=jax.ShapeDtypeStruct((NC, N_NODES, DEGW), jnp.float32),
        scratch_types=[
            pltpu.VMEM((CH,), jnp.int32),
            pltpu.VMEM((CH, DEGW), jnp.float32),
            pltpu.VMEM((RPT, DEGW), jnp.float32),
            pltpu.VMEM_SHARED((N_NODES, DEGW), jnp.float32),
        ],
    )
    spmm_kernel = pl.kernel(
        _spmm_body,
        mesh=mesh,
        compiler_params=params,
        out_type=jax.ShapeDtypeStruct((NC, N_NODES, D_FEAT), jnp.float32),
        scratch_types=[
            pltpu.VMEM((CH,), jnp.int32),
            pltpu.VMEM((CH,), jnp.int32),
            pltpu.VMEM((CH,), jnp.float32),
            pltpu.VMEM((CH, D_FEAT), jnp.float32),
            pltpu.VMEM((ZR, D_FEAT), jnp.float32),
            pltpu.VMEM_SHARED((N_NODES, D_FEAT), jnp.float32),
            pltpu.SemaphoreType.DMA,
        ],
    )
    return deg_kernel, spmm_kernel


_RB = 400
_GRID = N_NODES // _RB


def _prep_body(degp_ref, x_ref, dinv_ref, u0_ref):
    deg = degp_ref[0] + degp_ref[1]
    deg = jnp.where(deg < 0.5, deg + 1.0, deg)
    dinv = lax.rsqrt(deg)
    dinv_ref[...] = dinv
    u0_ref[...] = x_ref[...] * dinv[:, 0:1]


_prep = pl.pallas_call(
    _prep_body,
    grid=(_GRID,),
    in_specs=[
        pl.BlockSpec((NC, _RB, DEGW), lambda i: (0, i, 0)),
        pl.BlockSpec((_RB, D_FEAT), lambda i: (i, 0)),
    ],
    out_specs=[
        pl.BlockSpec((_RB, DEGW), lambda i: (i, 0)),
        pl.BlockSpec((_RB, D_FEAT), lambda i: (i, 0)),
    ],
    out_shape=[
        jax.ShapeDtypeStruct((N_NODES, DEGW), jnp.float32),
        jax.ShapeDtypeStruct((N_NODES, D_FEAT), jnp.float32),
    ],
)


def _combine_body(coef_ref, sp_ref, dinv_ref, yp_ref, yp2_ref, y_ref, u_ref):
    dinv = dinv_ref[...][:, 0:1]
    s = (sp_ref[0] + sp_ref[1]) * dinv
    y = coef_ref[0] * s + coef_ref[1] * yp_ref[...] + coef_ref[2] * yp2_ref[...]
    y_ref[...] = y
    u_ref[...] = y * dinv


_combine = pl.pallas_call(
    _combine_body,
    grid=(_GRID,),
    in_specs=[
        pl.BlockSpec(memory_space=pltpu.SMEM),
        pl.BlockSpec((NC, _RB, D_FEAT), lambda i: (0, i, 0)),
        pl.BlockSpec((_RB, DEGW), lambda i: (i, 0)),
        pl.BlockSpec((_RB, D_FEAT), lambda i: (i, 0)),
        pl.BlockSpec((_RB, D_FEAT), lambda i: (i, 0)),
    ],
    out_specs=[
        pl.BlockSpec((_RB, D_FEAT), lambda i: (i, 0)),
        pl.BlockSpec((_RB, D_FEAT), lambda i: (i, 0)),
    ],
    out_shape=[
        jax.ShapeDtypeStruct((N_NODES, D_FEAT), jnp.float32),
        jax.ShapeDtypeStruct((N_NODES, D_FEAT), jnp.float32),
    ],
)


def _coefficients(alphas):
    al = BASEALPHA * jnp.tanh(alphas)
    rml = R_HI - L_LO
    c1 = ((A_C - B_C) / 2 - (A_C + B_C + 2) / 2 * (L_LO + R_HI) / rml) * al[0]
    c2 = ((A_C + B_C + 2) / rml) * al[0]
    out = [jnp.stack([c2, c1, jnp.zeros_like(c1)])]
    for L in range(2, DEPTH + 1):
        coef_l = 2 * L * (L + A_C + B_C) * (2 * L - 2 + A_C + B_C)
        coef_lm1_1 = (2 * L + A_C + B_C - 1) * (2 * L + A_C + B_C) * (2 * L + A_C + B_C - 2)
        coef_lm1_2 = (2 * L + A_C + B_C - 1) * (A_C**2 - B_C**2)
        coef_lm2 = 2 * (L - 1 + A_C) * (L - 1 + B_C) * (2 * L + A_C + B_C)
        tmp1 = al[L - 1] * (coef_lm1_1 / coef_l)
        tmp2 = al[L - 1] * (coef_lm1_2 / coef_l)
        tmp3 = al[L - 1] * al[L - 2] * (coef_lm2 / coef_l)
        tmp1_2 = tmp1 * (2 / rml)
        tmp2_2 = tmp1 * ((R_HI + L_LO) / rml) + tmp2
        out.append(jnp.stack([tmp1_2, -tmp2_2, -tmp3]))
    return out


def kernel(x, edge_index, edge_attr, alphas):
    _deg_kernel, _spmm_kernel = _sc_kernels()
    row = edge_index[0]
    col = edge_index[1]
    degp = _deg_kernel(row)
    dinv16, u = _prep(degp, x)
    coefs = _coefficients(alphas)
    ys = [x]
    yprev2 = x
    for cf in coefs:
        sp = _spmm_kernel(u, row, col, edge_attr)
        y, u = _combine(cf, sp, dinv16, ys[-1], yprev2)
        yprev2 = ys[-1]
        ys.append(y)
    return jnp.stack(ys, axis=1)


# speedup vs baseline: 8.6606x; 1.5368x over previous
import functools

import jax
import jax.numpy as jnp
from jax import lax
from jax.experimental import pallas as pl
from jax.experimental.pallas import tpu as pltpu
from jax.experimental.pallas import tpu_sc as plsc

N_NODES = 10000
D_FEAT = 128
N_EDGES = 320000
DEPTH = 3
BASEALPHA = 1.0
A_C = 1.0
B_C = 1.0
L_LO = -1.0
R_HI = 1.0

NC = 2
NS = 16
NW = NC * NS
CH = 128
NCHG = N_EDGES // CH
NCH0 = NCHG // NW
REM = NCHG - NCH0 * NW
RPT = N_NODES // NS
ZR = 125
DEGW = 16


def _deg_body(row_hbm, out_hbm, rowv0, rowv1, onesb, zbuf, acc, ssem):
    cid = lax.axis_index("c")
    sid = lax.axis_index("s")
    wid = sid * NC + cid
    nch = NCH0 + jnp.where(wid < REM, 1, 0)

    def fill(i, carry):
        onesb[i, :] = jnp.full((DEGW,), 1.0, jnp.float32)
        return carry

    lax.fori_loop(0, CH, fill, 0)

    def zfill(i, carry):
        zbuf[i, :] = jnp.zeros((DEGW,), jnp.float32)
        return carry

    lax.fori_loop(0, RPT, zfill, 0)
    pltpu.sync_copy(zbuf, acc.at[pl.ds(sid * RPT, RPT)])
    plsc.subcore_barrier()

    rowv = (rowv0, rowv1)

    def cbase(jj):
        return (jj * NW + wid) * CH

    pltpu.sync_copy(row_hbm.at[pl.ds(cbase(0), CH)], rowv0)

    @pl.loop(0, NCH0 // 2)
    def _(t):
        for b in range(2):
            jj = 2 * t + b
            b1 = 1 - b

            @pl.when(jj >= 2)
            def _():
                pltpu.make_async_copy(onesb, acc.at[rowv[b]], ssem).wait()

            pltpu.async_copy(onesb, acc.at[rowv[b]], ssem, add=True)

            @pl.when(jj + 1 < nch)
            def _():
                pltpu.sync_copy(row_hbm.at[pl.ds(cbase(jj + 1), CH)], rowv[b1])

    pltpu.make_async_copy(onesb, acc.at[rowv0], ssem).wait()
    pltpu.make_async_copy(onesb, acc.at[rowv1], ssem).wait()

    @pl.when(wid < REM)
    def _():
        pltpu.sync_copy(onesb, acc.at[rowv0], add=True)

    plsc.subcore_barrier()
    pltpu.sync_copy(
        acc.at[pl.ds(sid * RPT, RPT)], out_hbm.at[cid, pl.ds(sid * RPT, RPT)]
    )


def _spmm_body(
    u_hbm, row_hbm, col_hbm, w_hbm, zer_hbm, out_hbm,
    colv0, colv1, rowv0, rowv1, wv0, wv1, rows0, rows1,
    acc, gsem, ssem,
):
    cid = lax.axis_index("c")
    sid = lax.axis_index("s")
    wid = sid * NC + cid
    nch = NCH0 + jnp.where(wid < REM, 1, 0)

    colv = (colv0, colv1)
    rowv = (rowv0, rowv1)
    wvs = (wv0, wv1)
    rows = (rows0, rows1)

    for q in range(RPT // ZR):
        pltpu.sync_copy(zer_hbm, acc.at[pl.ds(sid * RPT + q * ZR, ZR)])
    plsc.subcore_barrier()

    def cbase(jj):
        return (jj * NW + wid) * CH

    def do_scale(b):
        def scale(e, c2):
            bc = plsc.load_gather(wvs[b], [lax.broadcast(e, (16,))])
            for k in range(D_FEAT // 16):
                sl = pl.ds(k * 16, 16)
                rows[b][e, sl] = rows[b][e, sl] * bc
            return c2

        lax.fori_loop(0, CH, scale, 0, unroll=2)

    pltpu.sync_copy(col_hbm.at[pl.ds(cbase(0), CH)], colv0)
    pltpu.sync_copy(w_hbm.at[pl.ds(cbase(0), CH)], wv0)
    pltpu.async_copy(u_hbm.at[colv0], rows0, gsem)

    @pl.loop(0, NCH0 // 2)
    def _(t):
        for b in range(2):
            jj = 2 * t + b
            b1 = 1 - b
            pltpu.make_async_copy(u_hbm.at[colv[b]], rows[b], gsem).wait()

            @pl.when(jj >= 1)
            def _():
                pltpu.make_async_copy(rows[b1], acc.at[rowv[b1]], ssem).wait()

            @pl.when(jj + 1 < nch)
            def _():
                nb = cbase(jj + 1)
                pltpu.sync_copy(col_hbm.at[pl.ds(nb, CH)], colv[b1])
                pltpu.sync_copy(w_hbm.at[pl.ds(nb, CH)], wvs[b1])
                pltpu.async_copy(u_hbm.at[colv[b1]], rows[b1], gsem)

            pltpu.sync_copy(row_hbm.at[pl.ds(cbase(jj), CH)], rowv[b])
            do_scale(b)
            pltpu.async_copy(rows[b], acc.at[rowv[b]], ssem, add=True)

    pltpu.make_async_copy(rows1, acc.at[rowv1], ssem).wait()

    @pl.when(wid < REM)
    def _():
        pltpu.make_async_copy(u_hbm.at[colv0], rows0, gsem).wait()
        pltpu.sync_copy(row_hbm.at[pl.ds(cbase(NCH0), CH)], rowv0)
        do_scale(0)
        pltpu.sync_copy(rows0, acc.at[rowv0], add=True)

    plsc.subcore_barrier()
    for q in range(RPT // ZR):
        sl = pl.ds(sid * RPT + q * ZR, ZR)
        pltpu.sync_copy(acc.at[sl], out_hbm.at[cid, sl])


@functools.lru_cache(maxsize=None)
def _sc_kernels():
    mesh = plsc.VectorSubcoreMesh(
        core_axis_name="c", subcore_axis_name="s", num_cores=NC, num_subcores=NS
    )
    params = pltpu.CompilerParams(
        use_tc_tiling_on_sc=False, needs_layout_passes=False
    )
    deg_kernel = pl.kernel(
        _deg_body,
        mesh=mesh,
        compiler_params=params,
        out_type=jax.ShapeDtypeStruct((NC, N_NODES, DEGW), jnp.float32),
        scratch_types=[
            pltpu.VMEM((CH,), jnp.int32),
            pltpu.VMEM((CH,), jnp.int32),
            pltpu.VMEM((CH, DEGW), jnp.float32),
            pltpu.VMEM((RPT, DEGW), jnp.float32),
            pltpu.VMEM_SHARED((N_NODES, DEGW), jnp.float32),
            pltpu.SemaphoreType.DMA,
        ],
    )
    spmm_kernel = pl.kernel(
        _spmm_body,
        mesh=mesh,
        compiler_params=params,
        out_type=jax.ShapeDtypeStruct((NC, N_NODES, D_FEAT), jnp.float32),
        scratch_types=[
            pltpu.VMEM((CH,), jnp.int32),
            pltpu.VMEM((CH,), jnp.int32),
            pltpu.VMEM((CH,), jnp.int32),
            pltpu.VMEM((CH,), jnp.int32),
            pltpu.VMEM((CH,), jnp.float32),
            pltpu.VMEM((CH,), jnp.float32),
            pltpu.VMEM((CH, D_FEAT), jnp.float32),
            pltpu.VMEM((CH, D_FEAT), jnp.float32),
            pltpu.VMEM_SHARED((N_NODES, D_FEAT), jnp.float32),
            pltpu.SemaphoreType.DMA,
            pltpu.SemaphoreType.DMA,
        ],
    )
    return deg_kernel, spmm_kernel


_RB = 400
_GRID = N_NODES // _RB


def _prep_body(degp_ref, x_ref, dinv_ref, u0_ref):
    deg = degp_ref[0] + degp_ref[1]
    deg = jnp.where(deg < 0.5, deg + 1.0, deg)
    dinv = lax.rsqrt(deg)
    dinv_ref[...] = dinv
    u0_ref[...] = x_ref[...] * dinv[:, 0:1]


_prep = pl.pallas_call(
    _prep_body,
    grid=(_GRID,),
    in_specs=[
        pl.BlockSpec((NC, _RB, DEGW), lambda i: (0, i, 0)),
        pl.BlockSpec((_RB, D_FEAT), lambda i: (i, 0)),
    ],
    out_specs=[
        pl.BlockSpec((_RB, DEGW), lambda i: (i, 0)),
        pl.BlockSpec((_RB, D_FEAT), lambda i: (i, 0)),
    ],
    out_shape=[
        jax.ShapeDtypeStruct((N_NODES, DEGW), jnp.float32),
        jax.ShapeDtypeStruct((N_NODES, D_FEAT), jnp.float32),
    ],
)


def _combine_body(coef_ref, sp_ref, dinv_ref, yp_ref, yp2_ref, y_ref, u_ref):
    dinv = dinv_ref[...][:, 0:1]
    s = (sp_ref[0] + sp_ref[1]) * dinv
    y = coef_ref[0] * s + coef_ref[1] * yp_ref[...] + coef_ref[2] * yp2_ref[...]
    y_ref[...] = y
    u_ref[...] = y * dinv


_combine = pl.pallas_call(
    _combine_body,
    grid=(_GRID,),
    in_specs=[
        pl.BlockSpec(memory_space=pltpu.SMEM),
        pl.BlockSpec((NC, _RB, D_FEAT), lambda i: (0, i, 0)),
        pl.BlockSpec((_RB, DEGW), lambda i: (i, 0)),
        pl.BlockSpec((_RB, D_FEAT), lambda i: (i, 0)),
        pl.BlockSpec((_RB, D_FEAT), lambda i: (i, 0)),
    ],
    out_specs=[
        pl.BlockSpec((_RB, D_FEAT), lambda i: (i, 0)),
        pl.BlockSpec((_RB, D_FEAT), lambda i: (i, 0)),
    ],
    out_shape=[
        jax.ShapeDtypeStruct((N_NODES, D_FEAT), jnp.float32),
        jax.ShapeDtypeStruct((N_NODES, D_FEAT), jnp.float32),
    ],
)


def _coefficients(alphas):
    al = BASEALPHA * jnp.tanh(alphas)
    rml = R_HI - L_LO
    c1 = ((A_C - B_C) / 2 - (A_C + B_C + 2) / 2 * (L_LO + R_HI) / rml) * al[0]
    c2 = ((A_C + B_C + 2) / rml) * al[0]
    out = [jnp.stack([c2, c1, jnp.zeros_like(c1)])]
    for L in range(2, DEPTH + 1):
        coef_l = 2 * L * (L + A_C + B_C) * (2 * L - 2 + A_C + B_C)
        coef_lm1_1 = (2 * L + A_C + B_C - 1) * (2 * L + A_C + B_C) * (2 * L + A_C + B_C - 2)
        coef_lm1_2 = (2 * L + A_C + B_C - 1) * (A_C**2 - B_C**2)
        coef_lm2 = 2 * (L - 1 + A_C) * (L - 1 + B_C) * (2 * L + A_C + B_C)
        tmp1 = al[L - 1] * (coef_lm1_1 / coef_l)
        tmp2 = al[L - 1] * (coef_lm1_2 / coef_l)
        tmp3 = al[L - 1] * al[L - 2] * (coef_lm2 / coef_l)
        tmp1_2 = tmp1 * (2 / rml)
        tmp2_2 = tmp1 * ((R_HI + L_LO) / rml) + tmp2
        out.append(jnp.stack([tmp1_2, -tmp2_2, -tmp3]))
    return out


def kernel(x, edge_index, edge_attr, alphas):
    _deg_kernel, _spmm_kernel = _sc_kernels()
    row = edge_index[0]
    col = edge_index[1]
    degp = _deg_kernel(row)
    dinv16, u = _prep(degp, x)
    zer = jnp.zeros((ZR, D_FEAT), jnp.float32)
    coefs = _coefficients(alphas)
    ys = [x]
    yprev2 = x
    for cf in coefs:
        sp = _spmm_kernel(u, row, col, edge_attr, zer)
        y, u = _combine(cf, sp, dinv16, ys[-1], yprev2)
        yprev2 = ys[-1]
        ys.append(y)
    return jnp.stack(ys, axis=1)


# speedup vs baseline: 12.6517x; 1.4608x over previous
import functools

import jax
import jax.numpy as jnp
from jax import lax
from jax.experimental import pallas as pl
from jax.experimental.pallas import tpu as pltpu
from jax.experimental.pallas import tpu_sc as plsc

N_NODES = 10000
D_FEAT = 128
N_EDGES = 320000
DEPTH = 3
BASEALPHA = 1.0
A_C = 1.0
B_C = 1.0
L_LO = -1.0
R_HI = 1.0

NC = 2
NS = 16
NW = NC * NS
CH = 128
NCHG = N_EDGES // CH
NCH0 = NCHG // NW
REM = NCHG - NCH0 * NW
RPT = N_NODES // NS
ZR = 125
DEGW = 16
GRP = 26


def _deg_body(row2d_hbm, out_hbm, rv2, onesb, zbuf, acc, ssem):
    cid = lax.axis_index("c")
    sid = lax.axis_index("s")
    wid = sid * NC + cid
    first = wid * NCH0 + jnp.minimum(wid, REM)

    def fill(i, carry):
        onesb[i, :] = jnp.full((DEGW,), 1.0, jnp.float32)
        return carry

    lax.fori_loop(0, CH, fill, 0)

    def zfill(i, carry):
        zbuf[i, :] = jnp.zeros((DEGW,), jnp.float32)
        return carry

    lax.fori_loop(0, RPT, zfill, 0)
    pltpu.sync_copy(zbuf, acc.at[pl.ds(sid * RPT, RPT)])
    plsc.subcore_barrier()

    for g in range(NCH0 // GRP):
        gbase = first + g * GRP
        if g > 0:
            for l in range(GRP):
                pltpu.make_async_copy(onesb, acc.at[rv2.at[l]], ssem).wait()
        pltpu.sync_copy(row2d_hbm.at[pl.ds(gbase, GRP)], rv2)
        for l in range(GRP):
            pltpu.async_copy(onesb, acc.at[rv2.at[l]], ssem, add=True)

    for l in range(GRP):
        pltpu.make_async_copy(onesb, acc.at[rv2.at[l]], ssem).wait()

    @pl.when(wid < REM)
    def _():
        tb = first + NCH0
        pltpu.sync_copy(row2d_hbm.at[pl.ds(tb, 1)], rv2.at[pl.ds(0, 1)])
        pltpu.sync_copy(onesb, acc.at[rv2.at[0]], add=True)

    plsc.subcore_barrier()
    pltpu.sync_copy(
        acc.at[pl.ds(sid * RPT, RPT)], out_hbm.at[cid, pl.ds(sid * RPT, RPT)]
    )


def _spmm_body(
    u_hbm, row2d_hbm, col2d_hbm, w2d_hbm, zer_hbm, out_hbm,
    cv2, rv2, wv2, rows0, rows1, acc, gsem, ssem,
):
    cid = lax.axis_index("c")
    sid = lax.axis_index("s")
    wid = sid * NC + cid
    first = wid * NCH0 + jnp.minimum(wid, REM)
    rows = (rows0, rows1)

    for q in range(RPT // ZR):
        pltpu.sync_copy(zer_hbm, acc.at[pl.ds(sid * RPT + q * ZR, ZR)])
    plsc.subcore_barrier()

    def do_scale(b, l):
        def scale(e, c2):
            bc = plsc.load_gather(
                wv2, [lax.broadcast(l, (16,)), lax.broadcast(e, (16,))]
            )
            for k in range(D_FEAT // 16):
                sl = pl.ds(k * 16, 16)
                rows[b][e, sl] = rows[b][e, sl] * bc
            return c2

        lax.fori_loop(0, CH, scale, 0, unroll=2)

    for g in range(NCH0 // GRP):
        gbase = first + g * GRP
        if g > 0:
            pltpu.make_async_copy(rows1, acc.at[rv2.at[GRP - 1]], ssem).wait()
        pltpu.sync_copy(col2d_hbm.at[pl.ds(gbase, GRP)], cv2)
        pltpu.sync_copy(row2d_hbm.at[pl.ds(gbase, GRP)], rv2)
        pltpu.sync_copy(w2d_hbm.at[pl.ds(gbase, GRP)], wv2)
        pltpu.async_copy(u_hbm.at[cv2.at[0]], rows0, gsem)

        @pl.loop(0, GRP // 2)
        def _(p):
            l0 = 2 * p
            l1 = 2 * p + 1
            pltpu.make_async_copy(u_hbm.at[cv2.at[l0]], rows0, gsem).wait()

            @pl.when(p >= 1)
            def _():
                pltpu.make_async_copy(rows1, acc.at[rv2.at[l0 - 1]], ssem).wait()

            pltpu.async_copy(u_hbm.at[cv2.at[l1]], rows1, gsem)
            do_scale(0, l0)
            pltpu.async_copy(rows0, acc.at[rv2.at[l0]], ssem, add=True)

            pltpu.make_async_copy(u_hbm.at[cv2.at[l1]], rows1, gsem).wait()
            pltpu.make_async_copy(rows0, acc.at[rv2.at[l0]], ssem).wait()

            @pl.when(p < GRP // 2 - 1)
            def _():
                pltpu.async_copy(u_hbm.at[cv2.at[l1 + 1]], rows0, gsem)

            do_scale(1, l1)
            pltpu.async_copy(rows1, acc.at[rv2.at[l1]], ssem, add=True)

    pltpu.make_async_copy(rows1, acc.at[rv2.at[GRP - 1]], ssem).wait()

    @pl.when(wid < REM)
    def _():
        tb = first + NCH0
        pltpu.sync_copy(col2d_hbm.at[pl.ds(tb, 1)], cv2.at[pl.ds(0, 1)])
        pltpu.sync_copy(row2d_hbm.at[pl.ds(tb, 1)], rv2.at[pl.ds(0, 1)])
        pltpu.sync_copy(w2d_hbm.at[pl.ds(tb, 1)], wv2.at[pl.ds(0, 1)])
        pltpu.async_copy(u_hbm.at[cv2.at[0]], rows0, gsem).wait()
        do_scale(0, 0)
        pltpu.sync_copy(rows0, acc.at[rv2.at[0]], add=True)

    plsc.subcore_barrier()
    for q in range(RPT // ZR):
        sl = pl.ds(sid * RPT + q * ZR, ZR)
        pltpu.sync_copy(acc.at[sl], out_hbm.at[cid, sl])


@functools.lru_cache(maxsize=None)
def _sc_kernels():
    mesh = plsc.VectorSubcoreMesh(
        core_axis_name="c", subcore_axis_name="s", num_cores=NC, num_subcores=NS
    )
    params = pltpu.CompilerParams(
        use_tc_tiling_on_sc=False, needs_layout_passes=False
    )
    deg_kernel = pl.kernel(
        _deg_body,
        mesh=mesh,
        compiler_params=params,
        out_type=jax.ShapeDtypeStruct((NC, N_NODES, DEGW), jnp.float32),
        scratch_types=[
            pltpu.VMEM((GRP, CH), jnp.int32),
            pltpu.VMEM((CH, DEGW), jnp.float32),
            pltpu.VMEM((RPT, DEGW), jnp.float32),
            pltpu.VMEM_SHARED((N_NODES, DEGW), jnp.float32),
            pltpu.SemaphoreType.DMA,
        ],
    )
    spmm_kernel = pl.kernel(
        _spmm_body,
        mesh=mesh,
        compiler_params=params,
        out_type=jax.ShapeDtypeStruct((NC, N_NODES, D_FEAT), jnp.float32),
        scratch_types=[
            pltpu.VMEM((GRP, CH), jnp.int32),
            pltpu.VMEM((GRP, CH), jnp.int32),
            pltpu.VMEM((GRP, CH), jnp.float32),
            pltpu.VMEM((CH, D_FEAT), jnp.float32),
            pltpu.VMEM((CH, D_FEAT), jnp.float32),
            pltpu.VMEM_SHARED((N_NODES, D_FEAT), jnp.float32),
            pltpu.SemaphoreType.DMA,
            pltpu.SemaphoreType.DMA,
        ],
    )
    return deg_kernel, spmm_kernel


_RB = 400
_GRID = N_NODES // _RB


def _prep_body(degp_ref, x_ref, dinv_ref, u0_ref):
    deg = degp_ref[0] + degp_ref[1]
    deg = jnp.where(deg < 0.5, deg + 1.0, deg)
    dinv = lax.rsqrt(deg)
    dinv_ref[...] = dinv
    u0_ref[...] = x_ref[...] * dinv[:, 0:1]


_prep = pl.pallas_call(
    _prep_body,
    grid=(_GRID,),
    in_specs=[
        pl.BlockSpec((NC, _RB, DEGW), lambda i: (0, i, 0)),
        pl.BlockSpec((_RB, D_FEAT), lambda i: (i, 0)),
    ],
    out_specs=[
        pl.BlockSpec((_RB, DEGW), lambda i: (i, 0)),
        pl.BlockSpec((_RB, D_FEAT), lambda i: (i, 0)),
    ],
    out_shape=[
        jax.ShapeDtypeStruct((N_NODES, DEGW), jnp.float32),
        jax.ShapeDtypeStruct((N_NODES, D_FEAT), jnp.float32),
    ],
)


def _combine_body(coef_ref, sp_ref, dinv_ref, yp_ref, yp2_ref, y_ref, u_ref):
    dinv = dinv_ref[...][:, 0:1]
    s = (sp_ref[0] + sp_ref[1]) * dinv
    y = coef_ref[0] * s + coef_ref[1] * yp_ref[...] + coef_ref[2] * yp2_ref[...]
    y_ref[...] = y
    u_ref[...] = y * dinv


_combine = pl.pallas_call(
    _combine_body,
    grid=(_GRID,),
    in_specs=[
        pl.BlockSpec(memory_space=pltpu.SMEM),
        pl.BlockSpec((NC, _RB, D_FEAT), lambda i: (0, i, 0)),
        pl.BlockSpec((_RB, DEGW), lambda i: (i, 0)),
        pl.BlockSpec((_RB, D_FEAT), lambda i: (i, 0)),
        pl.BlockSpec((_RB, D_FEAT), lambda i: (i, 0)),
    ],
    out_specs=[
        pl.BlockSpec((_RB, D_FEAT), lambda i: (i, 0)),
        pl.BlockSpec((_RB, D_FEAT), lambda i: (i, 0)),
    ],
    out_shape=[
        jax.ShapeDtypeStruct((N_NODES, D_FEAT), jnp.float32),
        jax.ShapeDtypeStruct((N_NODES, D_FEAT), jnp.float32),
    ],
)


def _coefficients(alphas):
    al = BASEALPHA * jnp.tanh(alphas)
    rml = R_HI - L_LO
    c1 = ((A_C - B_C) / 2 - (A_C + B_C + 2) / 2 * (L_LO + R_HI) / rml) * al[0]
    c2 = ((A_C + B_C + 2) / rml) * al[0]
    out = [jnp.stack([c2, c1, jnp.zeros_like(c1)])]
    for L in range(2, DEPTH + 1):
        coef_l = 2 * L * (L + A_C + B_C) * (2 * L - 2 + A_C + B_C)
        coef_lm1_1 = (2 * L + A_C + B_C - 1) * (2 * L + A_C + B_C) * (2 * L + A_C + B_C - 2)
        coef_lm1_2 = (2 * L + A_C + B_C - 1) * (A_C**2 - B_C**2)
        coef_lm2 = 2 * (L - 1 + A_C) * (L - 1 + B_C) * (2 * L + A_C + B_C)
        tmp1 = al[L - 1] * (coef_lm1_1 / coef_l)
        tmp2 = al[L - 1] * (coef_lm1_2 / coef_l)
        tmp3 = al[L - 1] * al[L - 2] * (coef_lm2 / coef_l)
        tmp1_2 = tmp1 * (2 / rml)
        tmp2_2 = tmp1 * ((R_HI + L_LO) / rml) + tmp2
        out.append(jnp.stack([tmp1_2, -tmp2_2, -tmp3]))
    return out


def kernel(x, edge_index, edge_attr, alphas):
    _deg_kernel, _spmm_kernel = _sc_kernels()
    row2d = edge_index[0].reshape(NCHG, CH)
    col2d = edge_index[1].reshape(NCHG, CH)
    w2d = edge_attr.reshape(NCHG, CH)
    degp = _deg_kernel(row2d)
    dinv16, u = _prep(degp, x)
    zer = jnp.zeros((ZR, D_FEAT), jnp.float32)
    coefs = _coefficients(alphas)
    ys = [x]
    yprev2 = x
    for cf in coefs:
        sp = _spmm_kernel(u, row2d, col2d, w2d, zer)
        y, u = _combine(cf, sp, dinv16, ys[-1], yprev2)
        yprev2 = ys[-1]
        ys.append(y)
    return jnp.stack(ys, axis=1)


# speedup vs baseline: 13.2199x; 1.0449x over previous
import functools

import jax
import jax.numpy as jnp
from jax import lax
from jax.experimental import pallas as pl
from jax.experimental.pallas import tpu as pltpu
from jax.experimental.pallas import tpu_sc as plsc

N_NODES = 10000
D_FEAT = 128
N_EDGES = 320000
DEPTH = 3
BASEALPHA = 1.0
A_C = 1.0
B_C = 1.0
L_LO = -1.0
R_HI = 1.0

NC = 2
NS = 16
NW = NC * NS
CH = 64
NCHG = N_EDGES // CH
NCH0 = NCHG // NW
REM = NCHG - NCH0 * NW
RPT = N_NODES // NS
ZR = 125
DEGW = 16
GRP = 12


def _deg_body(row2d_hbm, out_hbm, rv2, onesb, zbuf, acc, ssem):
    cid = lax.axis_index("c")
    sid = lax.axis_index("s")
    wid = sid * NC + cid
    first = wid * NCH0 + jnp.minimum(wid, REM)

    def fill(i, carry):
        onesb[i, :] = jnp.full((DEGW,), 1.0, jnp.float32)
        return carry

    lax.fori_loop(0, CH, fill, 0)

    def zfill(i, carry):
        zbuf[i, :] = jnp.zeros((DEGW,), jnp.float32)
        return carry

    lax.fori_loop(0, RPT, zfill, 0)
    pltpu.sync_copy(zbuf, acc.at[pl.ds(sid * RPT, RPT)])
    plsc.subcore_barrier()

    for g in range(NCH0 // GRP):
        gbase = first + g * GRP
        if g > 0:
            for l in range(GRP):
                pltpu.make_async_copy(onesb, acc.at[rv2.at[l]], ssem).wait()
        pltpu.sync_copy(row2d_hbm.at[pl.ds(gbase, GRP)], rv2)
        for l in range(GRP):
            pltpu.async_copy(onesb, acc.at[rv2.at[l]], ssem, add=True)

    for l in range(GRP):
        pltpu.make_async_copy(onesb, acc.at[rv2.at[l]], ssem).wait()

    @pl.when(wid < REM)
    def _():
        tb = first + NCH0
        pltpu.sync_copy(row2d_hbm.at[pl.ds(tb, 1)], rv2.at[pl.ds(0, 1)])
        pltpu.sync_copy(onesb, acc.at[rv2.at[0]], add=True)

    plsc.subcore_barrier()
    pltpu.sync_copy(
        acc.at[pl.ds(sid * RPT, RPT)], out_hbm.at[cid, pl.ds(sid * RPT, RPT)]
    )


def _spmm_body(
    u_hbm, row2d_hbm, col2d_hbm, w2d_hbm, zer_hbm, out_hbm,
    cv2, rv2, wv2, rows0, rows1, rows2, rows3, acc, gsem, ssem,
):
    cid = lax.axis_index("c")
    sid = lax.axis_index("s")
    wid = sid * NC + cid
    first = wid * NCH0 + jnp.minimum(wid, REM)
    rows = (rows0, rows1, rows2, rows3)

    for q in range(RPT // ZR):
        pltpu.sync_copy(zer_hbm, acc.at[pl.ds(sid * RPT + q * ZR, ZR)])
    plsc.subcore_barrier()

    def do_scale(r, l):
        def scale(e, c2):
            bc = plsc.load_gather(
                wv2, [lax.broadcast(l, (16,)), lax.broadcast(e, (16,))]
            )
            for k in range(D_FEAT // 16):
                sl = pl.ds(k * 16, 16)
                rows[r][e, sl] = rows[r][e, sl] * bc
            return c2

        lax.fori_loop(0, CH, scale, 0, unroll=2)

    for g in range(NCH0 // GRP):
        gbase = first + g * GRP
        if g > 0:
            pltpu.make_async_copy(rows2, acc.at[rv2.at[GRP - 2]], ssem).wait()
            pltpu.make_async_copy(rows3, acc.at[rv2.at[GRP - 1]], ssem).wait()
        pltpu.sync_copy(col2d_hbm.at[pl.ds(gbase, GRP)], cv2)
        pltpu.sync_copy(row2d_hbm.at[pl.ds(gbase, GRP)], rv2)
        pltpu.sync_copy(w2d_hbm.at[pl.ds(gbase, GRP)], wv2)
        pltpu.async_copy(u_hbm.at[cv2.at[0]], rows0, gsem)
        pltpu.async_copy(u_hbm.at[cv2.at[1]], rows1, gsem)

        @pl.loop(0, GRP // 4)
        def _(q):
            for s in range(4):
                l = 4 * q + s
                r2 = (s + 2) % 4
                pltpu.make_async_copy(u_hbm.at[cv2.at[l]], rows[s], gsem).wait()

                @pl.when(l >= 2)
                def _():
                    pltpu.make_async_copy(
                        rows[r2], acc.at[rv2.at[l - 2]], ssem
                    ).wait()

                @pl.when(l + 2 < GRP)
                def _():
                    pltpu.async_copy(u_hbm.at[cv2.at[l + 2]], rows[r2], gsem)

                do_scale(s, l)
                pltpu.async_copy(rows[s], acc.at[rv2.at[l]], ssem, add=True)

    pltpu.make_async_copy(rows2, acc.at[rv2.at[GRP - 2]], ssem).wait()
    pltpu.make_async_copy(rows3, acc.at[rv2.at[GRP - 1]], ssem).wait()

    @pl.when(wid < REM)
    def _():
        tb = first + NCH0
        pltpu.sync_copy(col2d_hbm.at[pl.ds(tb, 1)], cv2.at[pl.ds(0, 1)])
        pltpu.sync_copy(row2d_hbm.at[pl.ds(tb, 1)], rv2.at[pl.ds(0, 1)])
        pltpu.sync_copy(w2d_hbm.at[pl.ds(tb, 1)], wv2.at[pl.ds(0, 1)])
        pltpu.async_copy(u_hbm.at[cv2.at[0]], rows0, gsem).wait()
        do_scale(0, 0)
        pltpu.sync_copy(rows0, acc.at[rv2.at[0]], add=True)

    plsc.subcore_barrier()
    for q in range(RPT // ZR):
        sl = pl.ds(sid * RPT + q * ZR, ZR)
        pltpu.sync_copy(acc.at[sl], out_hbm.at[cid, sl])


@functools.lru_cache(maxsize=None)
def _sc_kernels():
    mesh = plsc.VectorSubcoreMesh(
        core_axis_name="c", subcore_axis_name="s", num_cores=NC, num_subcores=NS
    )
    params = pltpu.CompilerParams(
        use_tc_tiling_on_sc=False, needs_layout_passes=False
    )
    deg_kernel = pl.kernel(
        _deg_body,
        mesh=mesh,
        compiler_params=params,
        out_type=jax.ShapeDtypeStruct((NC, N_NODES, DEGW), jnp.float32),
        scratch_types=[
            pltpu.VMEM((GRP, CH), jnp.int32),
            pltpu.VMEM((CH, DEGW), jnp.float32),
            pltpu.VMEM((RPT, DEGW), jnp.float32),
            pltpu.VMEM_SHARED((N_NODES, DEGW), jnp.float32),
            pltpu.SemaphoreType.DMA,
        ],
    )
    spmm_kernel = pl.kernel(
        _spmm_body,
        mesh=mesh,
        compiler_params=params,
        out_type=jax.ShapeDtypeStruct((NC, N_NODES, D_FEAT), jnp.float32),
        scratch_types=[
            pltpu.VMEM((GRP, CH), jnp.int32),
            pltpu.VMEM((GRP, CH), jnp.int32),
            pltpu.VMEM((GRP, CH), jnp.float32),
            pltpu.VMEM((CH, D_FEAT), jnp.float32),
            pltpu.VMEM((CH, D_FEAT), jnp.float32),
            pltpu.VMEM((CH, D_FEAT), jnp.float32),
            pltpu.VMEM((CH, D_FEAT), jnp.float32),
            pltpu.VMEM_SHARED((N_NODES, D_FEAT), jnp.float32),
            pltpu.SemaphoreType.DMA,
            pltpu.SemaphoreType.DMA,
        ],
    )
    return deg_kernel, spmm_kernel


_RB = 400
_GRID = N_NODES // _RB


def _prep_body(degp_ref, x_ref, dinv_ref, u0_ref):
    deg = degp_ref[0] + degp_ref[1]
    deg = jnp.where(deg < 0.5, deg + 1.0, deg)
    dinv = lax.rsqrt(deg)
    dinv_ref[...] = dinv
    u0_ref[...] = x_ref[...] * dinv[:, 0:1]


_prep = pl.pallas_call(
    _prep_body,
    grid=(_GRID,),
    in_specs=[
        pl.BlockSpec((NC, _RB, DEGW), lambda i: (0, i, 0)),
        pl.BlockSpec((_RB, D_FEAT), lambda i: (i, 0)),
    ],
    out_specs=[
        pl.BlockSpec((_RB, DEGW), lambda i: (i, 0)),
        pl.BlockSpec((_RB, D_FEAT), lambda i: (i, 0)),
    ],
    out_shape=[
        jax.ShapeDtypeStruct((N_NODES, DEGW), jnp.float32),
        jax.ShapeDtypeStruct((N_NODES, D_FEAT), jnp.float32),
    ],
)


def _combine_body(coef_ref, sp_ref, dinv_ref, yp_ref, yp2_ref, y_ref, u_ref):
    dinv = dinv_ref[...][:, 0:1]
    s = (sp_ref[0] + sp_ref[1]) * dinv
    y = coef_ref[0] * s + coef_ref[1] * yp_ref[...] + coef_ref[2] * yp2_ref[...]
    y_ref[...] = y
    u_ref[...] = y * dinv


_combine = pl.pallas_call(
    _combine_body,
    grid=(_GRID,),
    in_specs=[
        pl.BlockSpec(memory_space=pltpu.SMEM),
        pl.BlockSpec((NC, _RB, D_FEAT), lambda i: (0, i, 0)),
        pl.BlockSpec((_RB, DEGW), lambda i: (i, 0)),
        pl.BlockSpec((_RB, D_FEAT), lambda i: (i, 0)),
        pl.BlockSpec((_RB, D_FEAT), lambda i: (i, 0)),
    ],
    out_specs=[
        pl.BlockSpec((_RB, D_FEAT), lambda i: (i, 0)),
        pl.BlockSpec((_RB, D_FEAT), lambda i: (i, 0)),
    ],
    out_shape=[
        jax.ShapeDtypeStruct((N_NODES, D_FEAT), jnp.float32),
        jax.ShapeDtypeStruct((N_NODES, D_FEAT), jnp.float32),
    ],
)


def _coefficients(alphas):
    al = BASEALPHA * jnp.tanh(alphas)
    rml = R_HI - L_LO
    c1 = ((A_C - B_C) / 2 - (A_C + B_C + 2) / 2 * (L_LO + R_HI) / rml) * al[0]
    c2 = ((A_C + B_C + 2) / rml) * al[0]
    out = [jnp.stack([c2, c1, jnp.zeros_like(c1)])]
    for L in range(2, DEPTH + 1):
        coef_l = 2 * L * (L + A_C + B_C) * (2 * L - 2 + A_C + B_C)
        coef_lm1_1 = (2 * L + A_C + B_C - 1) * (2 * L + A_C + B_C) * (2 * L + A_C + B_C - 2)
        coef_lm1_2 = (2 * L + A_C + B_C - 1) * (A_C**2 - B_C**2)
        coef_lm2 = 2 * (L - 1 + A_C) * (L - 1 + B_C) * (2 * L + A_C + B_C)
        tmp1 = al[L - 1] * (coef_lm1_1 / coef_l)
        tmp2 = al[L - 1] * (coef_lm1_2 / coef_l)
        tmp3 = al[L - 1] * al[L - 2] * (coef_lm2 / coef_l)
        tmp1_2 = tmp1 * (2 / rml)
        tmp2_2 = tmp1 * ((R_HI + L_LO) / rml) + tmp2
        out.append(jnp.stack([tmp1_2, -tmp2_2, -tmp3]))
    return out


def kernel(x, edge_index, edge_attr, alphas):
    _deg_kernel, _spmm_kernel = _sc_kernels()
    row2d = edge_index[0].reshape(NCHG, CH)
    col2d = edge_index[1].reshape(NCHG, CH)
    w2d = edge_attr.reshape(NCHG, CH)
    degp = _deg_kernel(row2d)
    dinv16, u = _prep(degp, x)
    zer = jnp.zeros((ZR, D_FEAT), jnp.float32)
    coefs = _coefficients(alphas)
    ys = [x]
    yprev2 = x
    for cf in coefs:
        sp = _spmm_kernel(u, row2d, col2d, w2d, zer)
        y, u = _combine(cf, sp, dinv16, ys[-1], yprev2)
        yprev2 = ys[-1]
        ys.append(y)
    return jnp.stack(ys, axis=1)


# speedup vs baseline: 15.5877x; 1.1791x over previous
import functools

import jax
import jax.numpy as jnp
from jax import lax
from jax.experimental import pallas as pl
from jax.experimental.pallas import tpu as pltpu
from jax.experimental.pallas import tpu_sc as plsc

N_NODES = 10000
D_FEAT = 128
N_EDGES = 320000
DEPTH = 3
BASEALPHA = 1.0
A_C = 1.0
B_C = 1.0
L_LO = -1.0
R_HI = 1.0

NC = 2
NS = 16
NW = NC * NS
CH = 64
NCHG = N_EDGES // CH
NCH0 = NCHG // NW
REM = NCHG - NCH0 * NW
RPT = N_NODES // NS
ZR = 125
DEGW = 16
GRP = 52


def _deg_body(row2d_hbm, out_hbm, rv2, onesb, zbuf, acc, ssem):
    cid = lax.axis_index("c")
    sid = lax.axis_index("s")
    wid = sid * NC + cid
    first = wid * NCH0 + jnp.minimum(wid, REM)

    def fill(i, carry):
        onesb[i, :] = jnp.full((DEGW,), 1.0, jnp.float32)
        return carry

    lax.fori_loop(0, CH, fill, 0)

    def zfill(i, carry):
        zbuf[i, :] = jnp.zeros((DEGW,), jnp.float32)
        return carry

    lax.fori_loop(0, RPT, zfill, 0)
    pltpu.sync_copy(zbuf, acc.at[pl.ds(sid * RPT, RPT)])
    plsc.subcore_barrier()

    for g in range(NCH0 // GRP):
        gbase = first + g * GRP
        if g > 0:
            for l in range(GRP):
                pltpu.make_async_copy(onesb, acc.at[rv2.at[l]], ssem).wait()
        pltpu.sync_copy(row2d_hbm.at[pl.ds(gbase, GRP)], rv2)
        for l in range(GRP):
            pltpu.async_copy(onesb, acc.at[rv2.at[l]], ssem, add=True)

    for l in range(GRP):
        pltpu.make_async_copy(onesb, acc.at[rv2.at[l]], ssem).wait()

    @pl.when(wid < REM)
    def _():
        tb = first + NCH0
        pltpu.sync_copy(row2d_hbm.at[pl.ds(tb, 1)], rv2.at[pl.ds(0, 1)])
        pltpu.sync_copy(onesb, acc.at[rv2.at[0]], add=True)

    plsc.subcore_barrier()
    pltpu.sync_copy(
        acc.at[pl.ds(sid * RPT, RPT)], out_hbm.at[cid, pl.ds(sid * RPT, RPT)]
    )


def _spmm_body(
    u_hbm, row2d_hbm, col2d_hbm, w2d_hbm, zer_hbm, out_hbm,
    cv2, rv2, wv2, rows0, rows1, rows2, rows3, acc, gsem, ssem,
):
    cid = lax.axis_index("c")
    sid = lax.axis_index("s")
    wid = sid * NC + cid
    first = wid * NCH0 + jnp.minimum(wid, REM)
    rows = (rows0, rows1, rows2, rows3)

    for q in range(RPT // ZR):
        pltpu.sync_copy(zer_hbm, acc.at[pl.ds(sid * RPT + q * ZR, ZR)])
    plsc.subcore_barrier()

    def do_scale(r, l):
        def scale(e, c2):
            bc = plsc.load_gather(
                wv2, [lax.broadcast(l, (16,)), lax.broadcast(e, (16,))]
            )
            for k in range(D_FEAT // 16):
                sl = pl.ds(k * 16, 16)
                rows[r][e, sl] = rows[r][e, sl] * bc
            return c2

        lax.fori_loop(0, CH, scale, 0, unroll=2)

    for g in range(NCH0 // GRP):
        gbase = first + g * GRP
        if g > 0:
            pltpu.make_async_copy(rows2, acc.at[rv2.at[GRP - 2]], ssem).wait()
            pltpu.make_async_copy(rows3, acc.at[rv2.at[GRP - 1]], ssem).wait()
        pltpu.sync_copy(col2d_hbm.at[pl.ds(gbase, GRP)], cv2)
        pltpu.sync_copy(row2d_hbm.at[pl.ds(gbase, GRP)], rv2)
        pltpu.sync_copy(w2d_hbm.at[pl.ds(gbase, GRP)], wv2)
        pltpu.async_copy(u_hbm.at[cv2.at[0]], rows0, gsem)
        pltpu.async_copy(u_hbm.at[cv2.at[1]], rows1, gsem)

        @pl.loop(0, GRP // 4)
        def _(q):
            for s in range(4):
                l = 4 * q + s
                r2 = (s + 2) % 4
                pltpu.make_async_copy(u_hbm.at[cv2.at[l]], rows[s], gsem).wait()

                @pl.when(l >= 2)
                def _():
                    pltpu.make_async_copy(
                        rows[r2], acc.at[rv2.at[l - 2]], ssem
                    ).wait()

                @pl.when(l + 2 < GRP)
                def _():
                    pltpu.async_copy(u_hbm.at[cv2.at[l + 2]], rows[r2], gsem)

                do_scale(s, l)
                pltpu.async_copy(rows[s], acc.at[rv2.at[l]], ssem, add=True)

    pltpu.make_async_copy(rows2, acc.at[rv2.at[GRP - 2]], ssem).wait()
    pltpu.make_async_copy(rows3, acc.at[rv2.at[GRP - 1]], ssem).wait()

    @pl.when(wid < REM)
    def _():
        tb = first + NCH0
        pltpu.sync_copy(col2d_hbm.at[pl.ds(tb, 1)], cv2.at[pl.ds(0, 1)])
        pltpu.sync_copy(row2d_hbm.at[pl.ds(tb, 1)], rv2.at[pl.ds(0, 1)])
        pltpu.sync_copy(w2d_hbm.at[pl.ds(tb, 1)], wv2.at[pl.ds(0, 1)])
        pltpu.async_copy(u_hbm.at[cv2.at[0]], rows0, gsem).wait()
        do_scale(0, 0)
        pltpu.sync_copy(rows0, acc.at[rv2.at[0]], add=True)

    plsc.subcore_barrier()
    for q in range(RPT // ZR):
        sl = pl.ds(sid * RPT + q * ZR, ZR)
        pltpu.sync_copy(acc.at[sl], out_hbm.at[cid, sl])


@functools.lru_cache(maxsize=None)
def _sc_kernels():
    mesh = plsc.VectorSubcoreMesh(
        core_axis_name="c", subcore_axis_name="s", num_cores=NC, num_subcores=NS
    )
    params = pltpu.CompilerParams(
        use_tc_tiling_on_sc=False, needs_layout_passes=False
    )
    deg_kernel = pl.kernel(
        _deg_body,
        mesh=mesh,
        compiler_params=params,
        out_type=jax.ShapeDtypeStruct((NC, N_NODES, DEGW), jnp.float32),
        scratch_types=[
            pltpu.VMEM((GRP, CH), jnp.int32),
            pltpu.VMEM((CH, DEGW), jnp.float32),
            pltpu.VMEM((RPT, DEGW), jnp.float32),
            pltpu.VMEM_SHARED((N_NODES, DEGW), jnp.float32),
            pltpu.SemaphoreType.DMA,
        ],
    )
    spmm_kernel = pl.kernel(
        _spmm_body,
        mesh=mesh,
        compiler_params=params,
        out_type=jax.ShapeDtypeStruct((NC, N_NODES, D_FEAT), jnp.float32),
        scratch_types=[
            pltpu.VMEM((GRP, CH), jnp.int32),
            pltpu.VMEM((GRP, CH), jnp.int32),
            pltpu.VMEM((GRP, CH), jnp.float32),
            pltpu.VMEM((CH, D_FEAT), jnp.float32),
            pltpu.VMEM((CH, D_FEAT), jnp.float32),
            pltpu.VMEM((CH, D_FEAT), jnp.float32),
            pltpu.VMEM((CH, D_FEAT), jnp.float32),
            pltpu.VMEM_SHARED((N_NODES, D_FEAT), jnp.float32),
            pltpu.SemaphoreType.DMA,
            pltpu.SemaphoreType.DMA,
        ],
    )
    return deg_kernel, spmm_kernel


_RB = 400
_GRID = N_NODES // _RB


def _prep_body(degp_ref, x_ref, dinv_ref, u0_ref):
    deg = degp_ref[0] + degp_ref[1]
    deg = jnp.where(deg < 0.5, deg + 1.0, deg)
    dinv = lax.rsqrt(deg)
    dinv_ref[...] = dinv
    u0_ref[...] = x_ref[...] * dinv[:, 0:1]


_prep = pl.pallas_call(
    _prep_body,
    grid=(_GRID,),
    in_specs=[
        pl.BlockSpec((NC, _RB, DEGW), lambda i: (0, i, 0)),
        pl.BlockSpec((_RB, D_FEAT), lambda i: (i, 0)),
    ],
    out_specs=[
        pl.BlockSpec((_RB, DEGW), lambda i: (i, 0)),
        pl.BlockSpec((_RB, D_FEAT), lambda i: (i, 0)),
    ],
    out_shape=[
        jax.ShapeDtypeStruct((N_NODES, DEGW), jnp.float32),
        jax.ShapeDtypeStruct((N_NODES, D_FEAT), jnp.float32),
    ],
)


def _combine_body(coef_ref, sp_ref, dinv_ref, yp_ref, yp2_ref, y_ref, u_ref):
    dinv = dinv_ref[...][:, 0:1]
    s = (sp_ref[0] + sp_ref[1]) * dinv
    y = coef_ref[0] * s + coef_ref[1] * yp_ref[...] + coef_ref[2] * yp2_ref[...]
    y_ref[...] = y
    u_ref[...] = y * dinv


_combine = pl.pallas_call(
    _combine_body,
    grid=(_GRID,),
    in_specs=[
        pl.BlockSpec(memory_space=pltpu.SMEM),
        pl.BlockSpec((NC, _RB, D_FEAT), lambda i: (0, i, 0)),
        pl.BlockSpec((_RB, DEGW), lambda i: (i, 0)),
        pl.BlockSpec((_RB, D_FEAT), lambda i: (i, 0)),
        pl.BlockSpec((_RB, D_FEAT), lambda i: (i, 0)),
    ],
    out_specs=[
        pl.BlockSpec((_RB, D_FEAT), lambda i: (i, 0)),
        pl.BlockSpec((_RB, D_FEAT), lambda i: (i, 0)),
    ],
    out_shape=[
        jax.ShapeDtypeStruct((N_NODES, D_FEAT), jnp.float32),
        jax.ShapeDtypeStruct((N_NODES, D_FEAT), jnp.float32),
    ],
)


def _coefficients(alphas):
    al = BASEALPHA * jnp.tanh(alphas)
    rml = R_HI - L_LO
    c1 = ((A_C - B_C) / 2 - (A_C + B_C + 2) / 2 * (L_LO + R_HI) / rml) * al[0]
    c2 = ((A_C + B_C + 2) / rml) * al[0]
    out = [jnp.stack([c2, c1, jnp.zeros_like(c1)])]
    for L in range(2, DEPTH + 1):
        coef_l = 2 * L * (L + A_C + B_C) * (2 * L - 2 + A_C + B_C)
        coef_lm1_1 = (2 * L + A_C + B_C - 1) * (2 * L + A_C + B_C) * (2 * L + A_C + B_C - 2)
        coef_lm1_2 = (2 * L + A_C + B_C - 1) * (A_C**2 - B_C**2)
        coef_lm2 = 2 * (L - 1 + A_C) * (L - 1 + B_C) * (2 * L + A_C + B_C)
        tmp1 = al[L - 1] * (coef_lm1_1 / coef_l)
        tmp2 = al[L - 1] * (coef_lm1_2 / coef_l)
        tmp3 = al[L - 1] * al[L - 2] * (coef_lm2 / coef_l)
        tmp1_2 = tmp1 * (2 / rml)
        tmp2_2 = tmp1 * ((R_HI + L_LO) / rml) + tmp2
        out.append(jnp.stack([tmp1_2, -tmp2_2, -tmp3]))
    return out


def kernel(x, edge_index, edge_attr, alphas):
    _deg_kernel, _spmm_kernel = _sc_kernels()
    row2d = edge_index[0].reshape(NCHG, CH)
    col2d = edge_index[1].reshape(NCHG, CH)
    w2d = edge_attr.reshape(NCHG, CH)
    degp = _deg_kernel(row2d)
    dinv16, u = _prep(degp, x)
    zer = jnp.zeros((ZR, D_FEAT), jnp.float32)
    coefs = _coefficients(alphas)
    ys = [x]
    yprev2 = x
    for cf in coefs:
        sp = _spmm_kernel(u, row2d, col2d, w2d, zer)
        y, u = _combine(cf, sp, dinv16, ys[-1], yprev2)
        yprev2 = ys[-1]
        ys.append(y)
    return jnp.stack(ys, axis=1)


# speedup vs baseline: 16.1288x; 1.0347x over previous
import functools

import jax
import jax.numpy as jnp
from jax import lax
from jax.experimental import pallas as pl
from jax.experimental.pallas import tpu as pltpu
from jax.experimental.pallas import tpu_sc as plsc

N_NODES = 10000
D_FEAT = 128
N_EDGES = 320000
DEPTH = 3
BASEALPHA = 1.0
A_C = 1.0
B_C = 1.0
L_LO = -1.0
R_HI = 1.0

NC = 2
NS = 16
NW = NC * NS
CH = 64
NCHG = N_EDGES // CH
NCH0 = NCHG // NW
REM = NCHG - NCH0 * NW
RPT = N_NODES // NS
ZR = 125
DEGW = 16
GRP = 52


def _deg_body(row2d_hbm, out_hbm, rv2, onesb, zbuf, acc, ssem):
    cid = lax.axis_index("c")
    sid = lax.axis_index("s")
    wid = sid * NC + cid
    first = wid * NCH0 + jnp.minimum(wid, REM)

    def fill(i, carry):
        onesb[i, :] = jnp.full((DEGW,), 1.0, jnp.float32)
        return carry

    lax.fori_loop(0, CH, fill, 0)

    def zfill(i, carry):
        zbuf[i, :] = jnp.zeros((DEGW,), jnp.float32)
        return carry

    lax.fori_loop(0, RPT, zfill, 0)
    pltpu.sync_copy(zbuf, acc.at[pl.ds(sid * RPT, RPT)])
    plsc.subcore_barrier()

    for g in range(NCH0 // GRP):
        gbase = first + g * GRP
        if g > 0:
            for l in range(GRP):
                pltpu.make_async_copy(onesb, acc.at[rv2.at[l]], ssem).wait()
        pltpu.sync_copy(row2d_hbm.at[pl.ds(gbase, GRP)], rv2)
        for l in range(GRP):
            pltpu.async_copy(onesb, acc.at[rv2.at[l]], ssem, add=True)

    for l in range(GRP):
        pltpu.make_async_copy(onesb, acc.at[rv2.at[l]], ssem).wait()

    @pl.when(wid < REM)
    def _():
        tb = first + NCH0
        pltpu.sync_copy(row2d_hbm.at[pl.ds(tb, 1)], rv2.at[pl.ds(0, 1)])
        pltpu.sync_copy(onesb, acc.at[rv2.at[0]], add=True)

    plsc.subcore_barrier()
    pltpu.sync_copy(
        acc.at[pl.ds(sid * RPT, RPT)], out_hbm.at[cid, pl.ds(sid * RPT, RPT)]
    )


def _spmm_body(
    u_hbm, row2d_hbm, col2d_hbm, w2d_hbm, zer_hbm, out_hbm,
    cv2, rv2, wv2, rows0, rows1, rows2, rows3, acc, gsem, ssem,
):
    cid = lax.axis_index("c")
    sid = lax.axis_index("s")
    wid = sid * NC + cid
    first = wid * NCH0 + jnp.minimum(wid, REM)
    rows = (rows0, rows1, rows2, rows3)

    for q in range(RPT // ZR):
        pltpu.sync_copy(zer_hbm, acc.at[pl.ds(sid * RPT + q * ZR, ZR)])
    plsc.subcore_barrier()

    def do_scale(r, l):
        def scale(e, c2):
            bc = plsc.load_gather(
                wv2, [lax.broadcast(l, (16,)), lax.broadcast(e, (16,))]
            )
            for k in range(D_FEAT // 16):
                sl = pl.ds(k * 16, 16)
                rows[r][e, sl] = rows[r][e, sl] * bc
            return c2

        lax.fori_loop(0, CH, scale, 0, unroll=2)

    for g in range(NCH0 // GRP):
        gbase = first + g * GRP
        if g > 0:
            pltpu.make_async_copy(rows2, acc.at[rv2.at[GRP - 2]], ssem).wait()
            pltpu.make_async_copy(rows3, acc.at[rv2.at[GRP - 1]], ssem).wait()
        pltpu.sync_copy(col2d_hbm.at[pl.ds(gbase, GRP)], cv2)
        pltpu.sync_copy(row2d_hbm.at[pl.ds(gbase, GRP)], rv2)
        pltpu.sync_copy(w2d_hbm.at[pl.ds(gbase, GRP)], wv2)
        pltpu.async_copy(u_hbm.at[cv2.at[0]], rows0, gsem)
        pltpu.async_copy(u_hbm.at[cv2.at[1]], rows1, gsem)

        @pl.loop(0, GRP // 4)
        def _(q):
            for s in range(4):
                l = 4 * q + s
                r2 = (s + 2) % 4
                pltpu.make_async_copy(u_hbm.at[cv2.at[l]], rows[s], gsem).wait()

                @pl.when(l >= 2)
                def _():
                    pltpu.make_async_copy(
                        rows[r2], acc.at[rv2.at[l - 2]], ssem
                    ).wait()

                @pl.when(l + 2 < GRP)
                def _():
                    pltpu.async_copy(u_hbm.at[cv2.at[l + 2]], rows[r2], gsem)

                do_scale(s, l)
                pltpu.async_copy(rows[s], acc.at[rv2.at[l]], ssem, add=True)

    pltpu.make_async_copy(rows2, acc.at[rv2.at[GRP - 2]], ssem).wait()
    pltpu.make_async_copy(rows3, acc.at[rv2.at[GRP - 1]], ssem).wait()

    @pl.when(wid < REM)
    def _():
        tb = first + NCH0
        pltpu.sync_copy(col2d_hbm.at[pl.ds(tb, 1)], cv2.at[pl.ds(0, 1)])
        pltpu.sync_copy(row2d_hbm.at[pl.ds(tb, 1)], rv2.at[pl.ds(0, 1)])
        pltpu.sync_copy(w2d_hbm.at[pl.ds(tb, 1)], wv2.at[pl.ds(0, 1)])
        pltpu.async_copy(u_hbm.at[cv2.at[0]], rows0, gsem).wait()
        do_scale(0, 0)
        pltpu.sync_copy(rows0, acc.at[rv2.at[0]], add=True)

    plsc.subcore_barrier()
    for q in range(RPT // ZR):
        sl = pl.ds(sid * RPT + q * ZR, ZR)
        pltpu.sync_copy(acc.at[sl], out_hbm.at[cid, sl])


@functools.lru_cache(maxsize=None)
def _sc_kernels():
    mesh = plsc.VectorSubcoreMesh(
        core_axis_name="c", subcore_axis_name="s", num_cores=NC, num_subcores=NS
    )
    params = pltpu.CompilerParams(
        use_tc_tiling_on_sc=False, needs_layout_passes=False
    )
    deg_kernel = pl.kernel(
        _deg_body,
        mesh=mesh,
        compiler_params=params,
        out_type=jax.ShapeDtypeStruct((NC, N_NODES, DEGW), jnp.float32),
        scratch_types=[
            pltpu.VMEM((GRP, CH), jnp.int32),
            pltpu.VMEM((CH, DEGW), jnp.float32),
            pltpu.VMEM((RPT, DEGW), jnp.float32),
            pltpu.VMEM_SHARED((N_NODES, DEGW), jnp.float32),
            pltpu.SemaphoreType.DMA,
        ],
    )
    spmm_kernel = pl.kernel(
        _spmm_body,
        mesh=mesh,
        compiler_params=params,
        out_type=jax.ShapeDtypeStruct((NC, N_NODES, D_FEAT), jnp.float32),
        scratch_types=[
            pltpu.VMEM((GRP, CH), jnp.int32),
            pltpu.VMEM((GRP, CH), jnp.int32),
            pltpu.VMEM((GRP, CH), jnp.float32),
            pltpu.VMEM((CH, D_FEAT), jnp.float32),
            pltpu.VMEM((CH, D_FEAT), jnp.float32),
            pltpu.VMEM((CH, D_FEAT), jnp.float32),
            pltpu.VMEM((CH, D_FEAT), jnp.float32),
            pltpu.VMEM_SHARED((N_NODES, D_FEAT), jnp.float32),
            pltpu.SemaphoreType.DMA,
            pltpu.SemaphoreType.DMA,
        ],
    )
    return deg_kernel, spmm_kernel


_RB = 400
_GRID = N_NODES // _RB


def _prep_body(degp_ref, x_ref, dinv_ref, u0_ref):
    deg = degp_ref[0] + degp_ref[1]
    deg = jnp.where(deg < 0.5, deg + 1.0, deg)
    dinv = lax.rsqrt(deg)
    dinv_ref[...] = dinv
    u0_ref[...] = x_ref[...] * dinv[:, 0:1]


_prep = pl.pallas_call(
    _prep_body,
    grid=(_GRID,),
    in_specs=[
        pl.BlockSpec((NC, _RB, DEGW), lambda i: (0, i, 0)),
        pl.BlockSpec((_RB, D_FEAT), lambda i: (i, 0)),
    ],
    out_specs=[
        pl.BlockSpec((_RB, DEGW), lambda i: (i, 0)),
        pl.BlockSpec((_RB, D_FEAT), lambda i: (i, 0)),
    ],
    out_shape=[
        jax.ShapeDtypeStruct((N_NODES, DEGW), jnp.float32),
        jax.ShapeDtypeStruct((N_NODES, D_FEAT), jnp.float32),
    ],
)


def _combine_body(coef_ref, sp_ref, dinv_ref, yp_ref, yp2_ref, y_ref, u_ref):
    dinv = dinv_ref[...][:, 0:1]
    s = (sp_ref[0] + sp_ref[1]) * dinv
    y = coef_ref[0] * s + coef_ref[1] * yp_ref[...] + coef_ref[2] * yp2_ref[...]
    y_ref[...] = y
    u_ref[...] = y * dinv


_combine = pl.pallas_call(
    _combine_body,
    grid=(_GRID,),
    in_specs=[
        pl.BlockSpec(memory_space=pltpu.SMEM),
        pl.BlockSpec((NC, _RB, D_FEAT), lambda i: (0, i, 0)),
        pl.BlockSpec((_RB, DEGW), lambda i: (i, 0)),
        pl.BlockSpec((_RB, D_FEAT), lambda i: (i, 0)),
        pl.BlockSpec((_RB, D_FEAT), lambda i: (i, 0)),
    ],
    out_specs=[
        pl.BlockSpec((_RB, D_FEAT), lambda i: (i, 0)),
        pl.BlockSpec((_RB, D_FEAT), lambda i: (i, 0)),
    ],
    out_shape=[
        jax.ShapeDtypeStruct((N_NODES, D_FEAT), jnp.float32),
        jax.ShapeDtypeStruct((N_NODES, D_FEAT), jnp.float32),
    ],
)


def _final_body(coef_ref, sp_ref, dinv_ref, yp_ref, yp2_ref, x_ref, o4_ref):
    dinv = dinv_ref[...][:, 0:1]
    s = (sp_ref[0] + sp_ref[1]) * dinv
    y3 = coef_ref[0] * s + coef_ref[1] * yp_ref[...] + coef_ref[2] * yp2_ref[...]
    o4_ref[...] = jnp.stack(
        [x_ref[...], yp2_ref[...], yp_ref[...], y3], axis=1
    )


_final = pl.pallas_call(
    _final_body,
    grid=(_GRID,),
    in_specs=[
        pl.BlockSpec(memory_space=pltpu.SMEM),
        pl.BlockSpec((NC, _RB, D_FEAT), lambda i: (0, i, 0)),
        pl.BlockSpec((_RB, DEGW), lambda i: (i, 0)),
        pl.BlockSpec((_RB, D_FEAT), lambda i: (i, 0)),
        pl.BlockSpec((_RB, D_FEAT), lambda i: (i, 0)),
        pl.BlockSpec((_RB, D_FEAT), lambda i: (i, 0)),
    ],
    out_specs=pl.BlockSpec((_RB, DEPTH + 1, D_FEAT), lambda i: (i, 0, 0)),
    out_shape=jax.ShapeDtypeStruct((N_NODES, DEPTH + 1, D_FEAT), jnp.float32),
)


def _coefficients(alphas):
    al = BASEALPHA * jnp.tanh(alphas)
    rml = R_HI - L_LO
    c1 = ((A_C - B_C) / 2 - (A_C + B_C + 2) / 2 * (L_LO + R_HI) / rml) * al[0]
    c2 = ((A_C + B_C + 2) / rml) * al[0]
    out = [jnp.stack([c2, c1, jnp.zeros_like(c1)])]
    for L in range(2, DEPTH + 1):
        coef_l = 2 * L * (L + A_C + B_C) * (2 * L - 2 + A_C + B_C)
        coef_lm1_1 = (2 * L + A_C + B_C - 1) * (2 * L + A_C + B_C) * (2 * L + A_C + B_C - 2)
        coef_lm1_2 = (2 * L + A_C + B_C - 1) * (A_C**2 - B_C**2)
        coef_lm2 = 2 * (L - 1 + A_C) * (L - 1 + B_C) * (2 * L + A_C + B_C)
        tmp1 = al[L - 1] * (coef_lm1_1 / coef_l)
        tmp2 = al[L - 1] * (coef_lm1_2 / coef_l)
        tmp3 = al[L - 1] * al[L - 2] * (coef_lm2 / coef_l)
        tmp1_2 = tmp1 * (2 / rml)
        tmp2_2 = tmp1 * ((R_HI + L_LO) / rml) + tmp2
        out.append(jnp.stack([tmp1_2, -tmp2_2, -tmp3]))
    return out


def kernel(x, edge_index, edge_attr, alphas):
    _deg_kernel, _spmm_kernel = _sc_kernels()
    row2d = edge_index[0].reshape(NCHG, CH)
    col2d = edge_index[1].reshape(NCHG, CH)
    w2d = edge_attr.reshape(NCHG, CH)
    degp = _deg_kernel(row2d)
    dinv16, u = _prep(degp, x)
    zer = jnp.zeros((ZR, D_FEAT), jnp.float32)
    coefs = _coefficients(alphas)
    yp = x
    yp2 = x
    for cf in coefs[:-1]:
        sp = _spmm_kernel(u, row2d, col2d, w2d, zer)
        y, u = _combine(cf, sp, dinv16, yp, yp2)
        yp2 = yp
        yp = y
    sp = _spmm_kernel(u, row2d, col2d, w2d, zer)
    return _final(coefs[-1], sp, dinv16, yp, yp2, x)
